# baseline (speedup 1.0000x reference)
import math

import jax
import jax.numpy as jnp
from jax import lax
from jax.experimental import pallas as pl
from jax.experimental.pallas import tpu as pltpu

_INV_SQRT2 = 1.0 / math.sqrt(2.0)
_NUM_ANSWERS = 3129


def _pool_kernel(feat_ref, pos_ref, lang_ref, mf_ref, tail_ref, langm_ref):
    o = feat_ref.shape[1]
    bb = feat_ref.shape[0]
    mf_ref[...] = jnp.sum(feat_ref[...], axis=1) * (1.0 / o)
    mp = jnp.sum(pos_ref[...], axis=1) * (1.0 / o)
    tail_ref[...] = jnp.concatenate(
        [mp, jnp.zeros((bb, tail_ref.shape[1] - mp.shape[1]), jnp.float32)],
        axis=1)
    langm_ref[...] = jnp.mean(lang_ref[...], axis=1)


def _head_kernel(mf_ref, tail_ref, langm_ref, wvis_ref, wpool_ref, bpool_ref,
                 w1_ref, b1_ref, gamma_ref, beta_ref, w2_ref, b2_ref,
                 out_ref):
    xcat = jnp.concatenate([mf_ref[...], tail_ref[...]],
                           axis=1).astype(jnp.bfloat16)
    visn = jnp.dot(xcat, wvis_ref[...], preferred_element_type=jnp.float32)
    x = visn + langm_ref[...].astype(jnp.bfloat16).astype(jnp.float32)
    pooled = jnp.tanh(
        jnp.dot(x.astype(jnp.bfloat16), wpool_ref[...],
                preferred_element_type=jnp.float32) + bpool_ref[...])
    h = jnp.dot(pooled.astype(jnp.bfloat16), w1_ref[...],
                preferred_element_type=jnp.float32) + b1_ref[...]
    h = h * 0.5 * (1.0 + lax.erf(h * _INV_SQRT2))
    mu = jnp.mean(h, axis=-1, keepdims=True)
    var = jnp.mean((h - mu) ** 2, axis=-1, keepdims=True)
    hn = (h - mu) * lax.rsqrt(var + 1e-12) * gamma_ref[...] + beta_ref[...]
    out_ref[...] = (jnp.dot(hn.astype(jnp.bfloat16), w2_ref[...],
                            preferred_element_type=jnp.float32) + b2_ref[...])


def kernel(feat, pos, lang_emb, w_vis, wpool, bpool, w1, b1, gamma, beta,
           w2, b2):
    B, O, F = feat.shape
    S = lang_emb.shape[1]
    H = wpool.shape[0]
    H2 = w1.shape[1]
    Kp = w_vis.shape[0]
    Ap = w2.shape[1]
    TW = Kp - F

    BB = 8
    nk = B // (2 * BB)
    mf, tail, langm = pl.pallas_call(
        _pool_kernel,
        out_shape=(jax.ShapeDtypeStruct((B, F), jnp.float32),
                   jax.ShapeDtypeStruct((B, TW), jnp.float32),
                   jax.ShapeDtypeStruct((B, H), jnp.float32)),
        grid=(2, nk),
        in_specs=[
            pl.BlockSpec((BB, O, F), lambda i, k: (i * nk + k, 0, 0)),
            pl.BlockSpec((BB, O, 4), lambda i, k: (i * nk + k, 0, 0)),
            pl.BlockSpec((BB, S, H), lambda i, k: (i * nk + k, 0, 0)),
        ],
        out_specs=(pl.BlockSpec((BB, F), lambda i, k: (i * nk + k, 0)),
                   pl.BlockSpec((BB, TW), lambda i, k: (i * nk + k, 0)),
                   pl.BlockSpec((BB, H), lambda i, k: (i * nk + k, 0))),
        compiler_params=pltpu.CompilerParams(
            dimension_semantics=("parallel", "arbitrary"),
        ),
    )(feat, pos, lang_emb)

    ta = Ap // 2
    out = pl.pallas_call(
        _head_kernel,
        out_shape=jax.ShapeDtypeStruct((B, Ap), jnp.float32),
        grid=(2,),
        in_specs=[
            pl.BlockSpec((B, F), lambda j: (0, 0)),
            pl.BlockSpec((B, TW), lambda j: (0, 0)),
            pl.BlockSpec((B, H), lambda j: (0, 0)),
            pl.BlockSpec((Kp, H), lambda j: (0, 0)),
            pl.BlockSpec((H, H), lambda j: (0, 0)),
            pl.BlockSpec((1, H), lambda j: (0, 0)),
            pl.BlockSpec((H, H2), lambda j: (0, 0)),
            pl.BlockSpec((1, H2), lambda j: (0, 0)),
            pl.BlockSpec((1, H2), lambda j: (0, 0)),
            pl.BlockSpec((1, H2), lambda j: (0, 0)),
            pl.BlockSpec((H2, ta), lambda j: (0, j)),
            pl.BlockSpec((1, ta), lambda j: (0, j)),
        ],
        out_specs=pl.BlockSpec((B, ta), lambda j: (0, j)),
        compiler_params=pltpu.CompilerParams(
            dimension_semantics=("parallel",),
            vmem_limit_bytes=48 * 1024 * 1024,
        ),
    )(mf, tail, langm, w_vis, wpool, bpool, w1, b1, gamma, beta, w2, b2)

    return out[:, :_NUM_ANSWERS]


# speedup vs baseline: 2.5342x; 2.5342x over previous
import math

import jax
import jax.numpy as jnp
from jax import lax
from jax.experimental import pallas as pl
from jax.experimental.pallas import tpu as pltpu

_INV_SQRT2 = 1.0 / math.sqrt(2.0)
_NUM_ANSWERS = 3129


def _pool_kernel(feat_ref, pos_ref, lang_ref, mf_ref, tail_ref, langm_ref):
    o = feat_ref.shape[1]
    bb = feat_ref.shape[0]
    mf_ref[...] = jnp.sum(feat_ref[...], axis=1) * (1.0 / o)
    mp = jnp.sum(pos_ref[...], axis=1) * (1.0 / o)
    tail_ref[...] = jnp.concatenate(
        [mp, jnp.zeros((bb, tail_ref.shape[1] - mp.shape[1]), jnp.float32)],
        axis=1)
    langm_ref[...] = jnp.mean(lang_ref[...], axis=1)


def _head_kernel(mf_ref, tail_ref, langm_ref, wvis_ref, wpool_ref, bpool_ref,
                 w1_ref, b1_ref, gamma_ref, beta_ref, w2_ref, b2_ref,
                 out_ref):
    xcat = jnp.concatenate([mf_ref[...], tail_ref[...]],
                           axis=1).astype(jnp.bfloat16)
    visn = jnp.dot(xcat, wvis_ref[...], preferred_element_type=jnp.float32)
    x = visn + langm_ref[...].astype(jnp.bfloat16).astype(jnp.float32)
    pooled = jnp.tanh(
        jnp.dot(x.astype(jnp.bfloat16), wpool_ref[...],
                preferred_element_type=jnp.float32) + bpool_ref[...])
    h = jnp.dot(pooled.astype(jnp.bfloat16), w1_ref[...],
                preferred_element_type=jnp.float32) + b1_ref[...]
    h = h * 0.5 * (1.0 + lax.erf(h * _INV_SQRT2))
    mu = jnp.mean(h, axis=-1, keepdims=True)
    var = jnp.mean((h - mu) ** 2, axis=-1, keepdims=True)
    hn = (h - mu) * lax.rsqrt(var + 1e-12) * gamma_ref[...] + beta_ref[...]
    out_ref[...] = (jnp.dot(hn.astype(jnp.bfloat16), w2_ref[...],
                            preferred_element_type=jnp.float32) + b2_ref[...])


def kernel(feat, pos, lang_emb, w_vis, wpool, bpool, w1, b1, gamma, beta,
           w2, b2):
    B, O, F = feat.shape
    S = lang_emb.shape[1]
    H = wpool.shape[0]
    H2 = w1.shape[1]
    Kp = w_vis.shape[0]
    Ap = w2.shape[1]
    TW = Kp - F

    mf = jnp.mean(feat, axis=1)
    mp = jnp.mean(pos, axis=1)
    tail = jnp.pad(mp, ((0, 0), (0, TW - 4)))
    langm = jnp.mean(lang_emb, axis=1)

    ta = Ap // 2
    out = pl.pallas_call(
        _head_kernel,
        out_shape=jax.ShapeDtypeStruct((B, _NUM_ANSWERS), jnp.float32),
        grid=(2,),
        in_specs=[
            pl.BlockSpec((B, F), lambda j: (0, 0)),
            pl.BlockSpec((B, TW), lambda j: (0, 0)),
            pl.BlockSpec((B, H), lambda j: (0, 0)),
            pl.BlockSpec((Kp, H), lambda j: (0, 0)),
            pl.BlockSpec((H, H), lambda j: (0, 0)),
            pl.BlockSpec((1, H), lambda j: (0, 0)),
            pl.BlockSpec((H, H2), lambda j: (0, 0)),
            pl.BlockSpec((1, H2), lambda j: (0, 0)),
            pl.BlockSpec((1, H2), lambda j: (0, 0)),
            pl.BlockSpec((1, H2), lambda j: (0, 0)),
            pl.BlockSpec((H2, ta), lambda j: (0, j)),
            pl.BlockSpec((1, ta), lambda j: (0, j)),
        ],
        out_specs=pl.BlockSpec((B, ta), lambda j: (0, j)),
        compiler_params=pltpu.CompilerParams(
            dimension_semantics=("parallel",),
            vmem_limit_bytes=48 * 1024 * 1024,
        ),
    )(mf, tail, langm, w_vis, wpool, bpool, w1, b1, gamma, beta, w2, b2)

    return out


# speedup vs baseline: 2.6709x; 1.0539x over previous
import math

import jax
import jax.numpy as jnp
from jax import lax
from jax.experimental import pallas as pl
from jax.experimental.pallas import tpu as pltpu

_INV_SQRT2 = 1.0 / math.sqrt(2.0)
_NUM_ANSWERS = 3129


def _pool_kernel(feat_ref, pos_ref, lang_ref, mf_ref, tail_ref, langm_ref):
    o = feat_ref.shape[1]
    bb = feat_ref.shape[0]
    mf_ref[...] = jnp.sum(feat_ref[...], axis=1) * (1.0 / o)
    mp = jnp.sum(pos_ref[...], axis=1) * (1.0 / o)
    tail_ref[...] = jnp.concatenate(
        [mp, jnp.zeros((bb, tail_ref.shape[1] - mp.shape[1]), jnp.float32)],
        axis=1)
    langm_ref[...] = jnp.mean(lang_ref[...], axis=1)


def _head_kernel(mf_ref, tail_ref, langm_ref, wvis_ref, wpool_ref, bpool_ref,
                 w1_ref, b1_ref, gamma_ref, beta_ref, w2_ref, b2_ref,
                 out_ref):
    xcat = jnp.concatenate([mf_ref[...], tail_ref[...]],
                           axis=1).astype(jnp.bfloat16)
    visn = jnp.dot(xcat, wvis_ref[...], preferred_element_type=jnp.float32)
    x = visn + langm_ref[...].astype(jnp.bfloat16).astype(jnp.float32)
    pooled = jnp.tanh(
        jnp.dot(x.astype(jnp.bfloat16), wpool_ref[...],
                preferred_element_type=jnp.float32) + bpool_ref[...])
    h = jnp.dot(pooled.astype(jnp.bfloat16), w1_ref[...],
                preferred_element_type=jnp.float32) + b1_ref[...]
    h = h * 0.5 * (1.0 + lax.erf(h * _INV_SQRT2))
    mu = jnp.mean(h, axis=-1, keepdims=True)
    var = jnp.mean((h - mu) ** 2, axis=-1, keepdims=True)
    hn = (h - mu) * lax.rsqrt(var + 1e-12) * gamma_ref[...] + beta_ref[...]
    out_ref[...] = (jnp.dot(hn.astype(jnp.bfloat16), w2_ref[...],
                            preferred_element_type=jnp.float32) + b2_ref[...])


def kernel(feat, pos, lang_emb, w_vis, wpool, bpool, w1, b1, gamma, beta,
           w2, b2):
    B, O, F = feat.shape
    S = lang_emb.shape[1]
    H = wpool.shape[0]
    H2 = w1.shape[1]
    Kp = w_vis.shape[0]
    Ap = w2.shape[1]
    TW = Kp - F

    mf = jnp.mean(feat, axis=1)
    mp = jnp.mean(pos, axis=1)
    tail = jnp.pad(mp, ((0, 0), (0, TW - 4)))
    langm = jnp.mean(lang_emb, axis=1)

    ta = Ap
    out = pl.pallas_call(
        _head_kernel,
        out_shape=jax.ShapeDtypeStruct((B, _NUM_ANSWERS), jnp.float32),
        grid=(1,),
        in_specs=[
            pl.BlockSpec((B, F), lambda j: (0, 0)),
            pl.BlockSpec((B, TW), lambda j: (0, 0)),
            pl.BlockSpec((B, H), lambda j: (0, 0)),
            pl.BlockSpec((Kp, H), lambda j: (0, 0)),
            pl.BlockSpec((H, H), lambda j: (0, 0)),
            pl.BlockSpec((1, H), lambda j: (0, 0)),
            pl.BlockSpec((H, H2), lambda j: (0, 0)),
            pl.BlockSpec((1, H2), lambda j: (0, 0)),
            pl.BlockSpec((1, H2), lambda j: (0, 0)),
            pl.BlockSpec((1, H2), lambda j: (0, 0)),
            pl.BlockSpec((H2, ta), lambda j: (0, j)),
            pl.BlockSpec((1, ta), lambda j: (0, j)),
        ],
        out_specs=pl.BlockSpec((B, ta), lambda j: (0, j)),
        compiler_params=pltpu.CompilerParams(
            dimension_semantics=("parallel",),
            vmem_limit_bytes=48 * 1024 * 1024,
        ),
    )(mf, tail, langm, w_vis, wpool, bpool, w1, b1, gamma, beta, w2, b2)

    return out
